# baseline (speedup 1.0000x reference)
import functools

import jax
import jax.numpy as jnp
from jax import lax
from jax.experimental import pallas as pl
from jax.experimental.pallas import tpu as pltpu
from jax.experimental.pallas import tpu_sc as plsc

V = 100000
D = 64
B = 1024
C = 50

NC = 2
NS = 16
NW = NC * NS
BPW = B // NW
IDX_PER_W = BPW * C
GCHUNK = 128

_mesh = plsc.VectorSubcoreMesh(core_axis_name="c", subcore_axis_name="s")


@functools.partial(
    pl.kernel,
    out_type=jax.ShapeDtypeStruct((B, D), jnp.float32),
    mesh=_mesh,
    scratch_types=[
        pltpu.VMEM((IDX_PER_W,), jnp.int32),
        pltpu.VMEM((IDX_PER_W, D), jnp.float32),
        pltpu.VMEM((BPW, D), jnp.float32),
        pltpu.SemaphoreType.DMA,
    ],
    compiler_params=pltpu.CompilerParams(use_tc_tiling_on_sc=False),
)
def _gather_sum(cw_hbm, table_hbm, out_hbm, idx_v, rows_v, out_v, sem):
    wid = lax.axis_index("s") * NC + lax.axis_index("c")
    base = wid * IDX_PER_W

    pltpu.sync_copy(cw_hbm.at[pl.ds(base, IDX_PER_W)], idx_v)

    copies = []
    off = 0
    while off < IDX_PER_W:
        n = min(GCHUNK, IDX_PER_W - off)
        copies.append(
            pltpu.async_copy(
                table_hbm.at[idx_v.at[pl.ds(off, n)]],
                rows_v.at[pl.ds(off, n)],
                sem,
            )
        )
        off += n
    for cp in copies:
        cp.wait()

    def r_body(r, _):
        def c_body(c, accs):
            a0, a1, a2, a3 = accs
            row = r * C + c
            return (
                a0 + rows_v[row, pl.ds(0, 16)],
                a1 + rows_v[row, pl.ds(16, 16)],
                a2 + rows_v[row, pl.ds(32, 16)],
                a3 + rows_v[row, pl.ds(48, 16)],
            )

        z = jnp.zeros((16,), jnp.float32)
        a0, a1, a2, a3 = lax.fori_loop(0, C, c_body, (z, z, z, z))
        out_v[r, pl.ds(0, 16)] = a0
        out_v[r, pl.ds(16, 16)] = a1
        out_v[r, pl.ds(32, 16)] = a2
        out_v[r, pl.ds(48, 16)] = a3
        return 0

    lax.fori_loop(0, BPW, r_body, 0)

    pltpu.sync_copy(out_v, out_hbm.at[pl.ds(wid * BPW, BPW)])


VB = 2048


def _mm_body(sums_ref, w_ref, b_ref, out_ref):
    mean = sums_ref[...] * (1.0 / C)
    acc = lax.dot_general(
        mean,
        w_ref[...],
        (((1,), (1,)), ((), ())),
        preferred_element_type=jnp.float32,
    )
    out_ref[...] = acc + b_ref[...][None, :]


def _project(sums, W, b):
    nvb = pl.cdiv(V, VB)
    return pl.pallas_call(
        _mm_body,
        grid=(nvb,),
        in_specs=[
            pl.BlockSpec((B, D), lambda j: (0, 0)),
            pl.BlockSpec((VB, D), lambda j: (j, 0)),
            pl.BlockSpec((VB,), lambda j: (j,)),
        ],
        out_specs=pl.BlockSpec((B, VB), lambda j: (0, j)),
        out_shape=jax.ShapeDtypeStruct((B, V), jnp.float32),
        compiler_params=pltpu.CompilerParams(
            dimension_semantics=("arbitrary",),
        ),
    )(sums, W, b)


def kernel(context_words, emb_table, W, b):
    cw_flat = context_words.reshape(-1).astype(jnp.int32)
    sums = _gather_sum(cw_flat, emb_table)
    return _project(sums, W, b)


# speedup vs baseline: 2.6982x; 2.6982x over previous
import functools

import jax
import jax.numpy as jnp
from jax import lax
from jax.experimental import pallas as pl
from jax.experimental.pallas import tpu as pltpu
from jax.experimental.pallas import tpu_sc as plsc

V = 100000
D = 64
B = 1024
C = 50

NC = 2
NS = 16
NW = NC * NS
BPW = B // NW
IDX_PER_W = BPW * C
GCHUNK = 128

_mesh = plsc.VectorSubcoreMesh(core_axis_name="c", subcore_axis_name="s")


@functools.partial(
    pl.kernel,
    out_type=jax.ShapeDtypeStruct((B, D), jnp.float32),
    mesh=_mesh,
    scratch_types=[
        pltpu.VMEM((IDX_PER_W,), jnp.int32),
        pltpu.VMEM((IDX_PER_W, D), jnp.float32),
        pltpu.VMEM((BPW, D), jnp.float32),
        pltpu.SemaphoreType.DMA,
    ],
    compiler_params=pltpu.CompilerParams(use_tc_tiling_on_sc=False),
)
def _gather_mean(cw_hbm, table_hbm, out_hbm, idx_v, rows_v, out_v, sem):
    wid = lax.axis_index("s") * NC + lax.axis_index("c")
    base = wid * IDX_PER_W

    pltpu.sync_copy(cw_hbm.at[pl.ds(base, IDX_PER_W)], idx_v)

    copies = []
    off = 0
    while off < IDX_PER_W:
        n = min(GCHUNK, IDX_PER_W - off)
        copies.append(
            pltpu.async_copy(
                table_hbm.at[idx_v.at[pl.ds(off, n)]],
                rows_v.at[pl.ds(off, n)],
                sem,
            )
        )
        off += n
    for cp in copies:
        cp.wait()

    def r_body(r, _):
        def c_body(c, accs):
            a0, a1, a2, a3 = accs
            row = r * C + c
            return (
                a0 + rows_v[row, pl.ds(0, 16)],
                a1 + rows_v[row, pl.ds(16, 16)],
                a2 + rows_v[row, pl.ds(32, 16)],
                a3 + rows_v[row, pl.ds(48, 16)],
            )

        z = jnp.zeros((16,), jnp.float32)
        a0, a1, a2, a3 = lax.fori_loop(0, C, c_body, (z, z, z, z))
        inv = jnp.float32(1.0 / C)
        out_v[r, pl.ds(0, 16)] = a0 * inv
        out_v[r, pl.ds(16, 16)] = a1 * inv
        out_v[r, pl.ds(32, 16)] = a2 * inv
        out_v[r, pl.ds(48, 16)] = a3 * inv
        return 0

    lax.fori_loop(0, BPW, r_body, 0)

    pltpu.sync_copy(out_v, out_hbm.at[pl.ds(wid * BPW, BPW)])


VB = 2048


def _mm_body(mean_ref, wt_ref, b_ref, out_ref):
    acc = lax.dot_general(
        wt_ref[...],
        mean_ref[...],
        (((0,), (1,)), ((), ())),
        preferred_element_type=jnp.float32,
    )
    out_ref[...] = acc + b_ref[...][:, None]


def _project_t(mean, Wt, b):
    nvb = pl.cdiv(V, VB)
    return pl.pallas_call(
        _mm_body,
        grid=(nvb,),
        in_specs=[
            pl.BlockSpec((B, D), lambda j: (0, 0)),
            pl.BlockSpec((D, VB), lambda j: (0, j)),
            pl.BlockSpec((VB,), lambda j: (j,)),
        ],
        out_specs=pl.BlockSpec((VB, B), lambda j: (j, 0)),
        out_shape=jax.ShapeDtypeStruct((V, B), jnp.float32),
        compiler_params=pltpu.CompilerParams(
            dimension_semantics=("arbitrary",),
        ),
    )(mean, Wt, b)


def kernel(context_words, emb_table, W, b):
    cw_flat = context_words.reshape(-1).astype(jnp.int32)
    mean = _gather_mean(cw_flat, emb_table)
    out_t = _project_t(mean, W.T, b)
    return out_t.T


# speedup vs baseline: 3.2158x; 1.1918x over previous
import functools

import jax
import jax.numpy as jnp
from jax import lax
from jax.experimental import pallas as pl
from jax.experimental.pallas import tpu as pltpu
from jax.experimental.pallas import tpu_sc as plsc

V = 100000
D = 64
B = 1024
C = 50

NC = 2
NS = 16
NW = NC * NS
DPW = D // NW
BBLK = 128
NBBLK = B // BBLK
CWCHUNK = BBLK * C

_mesh = plsc.VectorSubcoreMesh(core_axis_name="c", subcore_axis_name="s")


@functools.partial(
    pl.kernel,
    out_type=jax.ShapeDtypeStruct((D, B), jnp.float32),
    mesh=_mesh,
    scratch_types=[
        pltpu.VMEM((V,), jnp.float32),
        pltpu.VMEM((CWCHUNK,), jnp.int32),
        pltpu.VMEM((B,), jnp.float32),
    ],
    compiler_params=pltpu.CompilerParams(needs_layout_passes=False),
)
def _gather_mean_t(cw_hbm, embt_hbm, out_hbm, row_v, cw_v, orow_v):
    wid = lax.axis_index("s") * NC + lax.axis_index("c")
    inv = jnp.float32(1.0 / C)
    lane = lax.iota(jnp.int32, 16)
    lane_c = lane * C

    for p in range(DPW):
        d = wid * DPW + p
        pltpu.sync_copy(embt_hbm.at[d], row_v)

        for bblk in range(NBBLK):
            pltpu.sync_copy(
                cw_hbm.at[pl.ds(bblk * CWCHUNK, CWCHUNK)], cw_v
            )

            def c_body(c, accs):
                new = []
                for g in range(BBLK // 16):
                    idx_pos = lane_c + (g * 16 * C + c)
                    idx16 = plsc.load_gather(cw_v, [idx_pos])
                    vals = plsc.load_gather(row_v, [idx16])
                    new.append(accs[g] + vals)
                return tuple(new)

            z = jnp.zeros((16,), jnp.float32)
            accs = lax.fori_loop(0, C, c_body, (z,) * (BBLK // 16))
            for g in range(BBLK // 16):
                orow_v[pl.ds(bblk * BBLK + g * 16, 16)] = accs[g] * inv

        pltpu.sync_copy(orow_v, out_hbm.at[d])


VB = 2048


def _mm_body(meant_ref, wt_ref, b_ref, out_ref):
    acc = lax.dot_general(
        wt_ref[...],
        meant_ref[...],
        (((0,), (0,)), ((), ())),
        preferred_element_type=jnp.float32,
    )
    out_ref[...] = acc + b_ref[...][:, None]


def _project_t(mean_t, Wt, b):
    nvb = pl.cdiv(V, VB)
    return pl.pallas_call(
        _mm_body,
        grid=(nvb,),
        in_specs=[
            pl.BlockSpec((D, B), lambda j: (0, 0)),
            pl.BlockSpec((D, VB), lambda j: (0, j)),
            pl.BlockSpec((VB,), lambda j: (j,)),
        ],
        out_specs=pl.BlockSpec((VB, B), lambda j: (j, 0)),
        out_shape=jax.ShapeDtypeStruct((V, B), jnp.float32),
        compiler_params=pltpu.CompilerParams(
            dimension_semantics=("arbitrary",),
        ),
    )(mean_t, Wt, b)


def kernel(context_words, emb_table, W, b):
    cw_flat = context_words.reshape(-1).astype(jnp.int32)
    mean_t = _gather_mean_t(cw_flat, emb_table.T)
    out_t = _project_t(mean_t, W.T, b)
    return out_t.T


# speedup vs baseline: 3.2424x; 1.0083x over previous
import functools

import jax
import jax.numpy as jnp
from jax import lax
from jax.experimental import pallas as pl
from jax.experimental.pallas import tpu as pltpu
from jax.experimental.pallas import tpu_sc as plsc

V = 100000
D = 64
B = 1024
C = 50

NC = 2
NS = 16
NW = NC * NS
DPW = D // NW
BBLK = 128
NBBLK = B // BBLK
CWCHUNK = BBLK * C

_mesh = plsc.VectorSubcoreMesh(core_axis_name="c", subcore_axis_name="s")


@functools.partial(
    pl.kernel,
    out_type=jax.ShapeDtypeStruct((D, B), jnp.float32),
    mesh=_mesh,
    scratch_types=[
        pltpu.VMEM((V,), jnp.float32),
        pltpu.VMEM((CWCHUNK,), jnp.int32),
        pltpu.VMEM((B,), jnp.float32),
    ],
    compiler_params=pltpu.CompilerParams(needs_layout_passes=False),
)
def _gather_mean_t(cw_hbm, embt_hbm, out_hbm, row_v, cw_v, orow_v):
    wid = lax.axis_index("s") * NC + lax.axis_index("c")
    inv = jnp.float32(1.0 / C)
    lane = lax.iota(jnp.int32, 16)
    lane_c = lane * C

    for p in range(DPW):
        d = wid * DPW + p
        pltpu.sync_copy(embt_hbm.at[d], row_v)

        for bblk in range(NBBLK):
            pltpu.sync_copy(
                cw_hbm.at[pl.ds(bblk * CWCHUNK, CWCHUNK)], cw_v
            )

            def c_body(c, accs):
                new = []
                for g in range(BBLK // 16):
                    idx_pos = lane_c + (g * 16 * C + c)
                    idx16 = plsc.load_gather(cw_v, [idx_pos])
                    vals = plsc.load_gather(row_v, [idx16])
                    new.append(accs[g] + vals)
                return tuple(new)

            z = jnp.zeros((16,), jnp.float32)
            accs = lax.fori_loop(0, C, c_body, (z,) * (BBLK // 16))
            for g in range(BBLK // 16):
                orow_v[pl.ds(bblk * BBLK + g * 16, 16)] = accs[g] * inv

        pltpu.sync_copy(orow_v, out_hbm.at[d])


VB = 4096


def _mm_body(meant_ref, wt_ref, b_ref, out_ref):
    acc = lax.dot_general(
        wt_ref[...],
        meant_ref[...],
        (((0,), (0,)), ((), ())),
        preferred_element_type=jnp.float32,
    )
    out_ref[...] = acc + b_ref[...][:, None]


def _project_t(mean_t, Wt, b):
    nvb = pl.cdiv(V, VB)
    return pl.pallas_call(
        _mm_body,
        grid=(nvb,),
        in_specs=[
            pl.BlockSpec((D, B), lambda j: (0, 0)),
            pl.BlockSpec((D, VB), lambda j: (0, j)),
            pl.BlockSpec((VB,), lambda j: (j,)),
        ],
        out_specs=pl.BlockSpec((VB, B), lambda j: (j, 0)),
        out_shape=jax.ShapeDtypeStruct((V, B), jnp.float32),
        compiler_params=pltpu.CompilerParams(
            dimension_semantics=("arbitrary",),
        ),
    )(mean_t, Wt, b)


def kernel(context_words, emb_table, W, b):
    cw_flat = context_words.reshape(-1).astype(jnp.int32)
    mean_t = _gather_mean_t(cw_flat, emb_table.T)
    out_t = _project_t(mean_t, W.T, b)
    return out_t.T


# speedup vs baseline: 3.3578x; 1.0356x over previous
import functools

import jax
import jax.numpy as jnp
from jax import lax
from jax.experimental import pallas as pl
from jax.experimental.pallas import tpu as pltpu
from jax.experimental.pallas import tpu_sc as plsc

V = 100000
D = 64
B = 1024
C = 50

NC = 2
NS = 16
NW = NC * NS
DPW = D // NW
BBLK = 128
NBBLK = B // BBLK
CWCHUNK = BBLK * C

_mesh = plsc.VectorSubcoreMesh(core_axis_name="c", subcore_axis_name="s")


@functools.partial(
    pl.kernel,
    out_type=jax.ShapeDtypeStruct((D, B), jnp.float32),
    mesh=_mesh,
    scratch_types=[
        pltpu.VMEM((V,), jnp.float32),
        pltpu.VMEM((C, BBLK), jnp.int32),
        pltpu.VMEM((C, BBLK), jnp.int32),
        pltpu.VMEM((B,), jnp.float32),
        pltpu.SemaphoreType.DMA,
        pltpu.SemaphoreType.DMA,
        pltpu.SemaphoreType.DMA,
    ],
    compiler_params=pltpu.CompilerParams(needs_layout_passes=False),
)
def _gather_mean_t(cwt_hbm, embt_hbm, out_hbm, row_v, cw_a, cw_b, orow_v,
                   sem_row, sem_a, sem_b):
    wid = lax.axis_index("s") * NC + lax.axis_index("c")
    inv = jnp.float32(1.0 / C)
    bufs = (cw_a, cw_b)
    sems = (sem_a, sem_b)

    def fire_cw(step):
        return pltpu.async_copy(
            cwt_hbm.at[:, pl.ds((step % NBBLK) * BBLK, BBLK)],
            bufs[step % 2],
            sems[step % 2],
        )

    nsteps = DPW * NBBLK
    row_cp = pltpu.async_copy(embt_hbm.at[wid * DPW], row_v, sem_row)
    pend = {0: fire_cw(0), 1: fire_cw(1)}

    for p in range(DPW):
        d = wid * DPW + p
        row_cp.wait()
        for bblk in range(NBBLK):
            i = p * NBBLK + bblk
            pend.pop(i).wait()
            cw_v = bufs[i % 2]

            def c_body(c, accs):
                new = []
                for g in range(BBLK // 16):
                    idx16 = cw_v[c, pl.ds(g * 16, 16)]
                    vals = plsc.load_gather(row_v, [idx16])
                    new.append(accs[g] + vals)
                return tuple(new)

            z = jnp.zeros((16,), jnp.float32)
            accs = lax.fori_loop(0, C, c_body, (z,) * (BBLK // 16))
            for g in range(BBLK // 16):
                orow_v[pl.ds(bblk * BBLK + g * 16, 16)] = accs[g] * inv

            if i + 2 < nsteps:
                pend[i + 2] = fire_cw(i + 2)
            if i == NBBLK - 1 and p + 1 < DPW:
                row_cp = pltpu.async_copy(
                    embt_hbm.at[wid * DPW + p + 1], row_v, sem_row
                )

        pltpu.sync_copy(orow_v, out_hbm.at[d])


VB = 4096


def _mm_body(meant_ref, wt_ref, b_ref, out_ref):
    acc = lax.dot_general(
        wt_ref[...],
        meant_ref[...],
        (((0,), (0,)), ((), ())),
        preferred_element_type=jnp.float32,
    )
    out_ref[...] = acc + b_ref[...][:, None]


def _project_t(mean_t, Wt, b):
    nvb = pl.cdiv(V, VB)
    return pl.pallas_call(
        _mm_body,
        grid=(nvb,),
        in_specs=[
            pl.BlockSpec((D, B), lambda j: (0, 0)),
            pl.BlockSpec((D, VB), lambda j: (0, j)),
            pl.BlockSpec((VB,), lambda j: (j,)),
        ],
        out_specs=pl.BlockSpec((VB, B), lambda j: (j, 0)),
        out_shape=jax.ShapeDtypeStruct((V, B), jnp.float32),
        compiler_params=pltpu.CompilerParams(
            dimension_semantics=("arbitrary",),
        ),
    )(mean_t, Wt, b)


def kernel(context_words, emb_table, W, b):
    cw_t = context_words.T.astype(jnp.int32)
    mean_t = _gather_mean_t(cw_t, emb_table.T)
    out_t = _project_t(mean_t, W.T, b)
    return out_t.T
